# baseline (speedup 1.0000x reference)
import functools

import jax
import jax.numpy as jnp
from jax import lax
from jax.experimental import pallas as pl
from jax.experimental.pallas import tpu as pltpu
from jax.experimental.pallas import tpu_sc as plsc

FC = 256
N = 10000
E = 160000
G = 64
LAYERS = 3
VMIN, VMAX = -4.0, 4.0

NP = 10240
RB = 1024
EB = 640
CH = 128
NCH = E // CH
CHS = 640
NCHS = E // CHS
NC, NS = 2, 16
NW = NC * NS

_tc_call = pl.pallas_call


def _silu(v):
    return v * (1.0 / (1.0 + jnp.exp(-v)))


def _pack2(a, b):
    ai = lax.bitcast_convert_type(a, jnp.int32) + 0x8000
    bi = lax.bitcast_convert_type(b, jnp.int32) + 0x8000
    return (ai & jnp.int32(-65536)) | (lax.shift_right_logical(bi, 16) & 0xFFFF)


def _unpack2(x):
    a = lax.bitcast_convert_type(x & jnp.int32(-65536), jnp.float32)
    b = lax.bitcast_convert_type(lax.shift_left(x, 16), jnp.float32)
    return a, b


def _sigmoid(v):
    return 1.0 / (1.0 + jnp.exp(-v))


def _bdot(a, w_ref):
    return jnp.dot(a.astype(jnp.bfloat16), w_ref[...],
                   preferred_element_type=jnp.float32)


def _edgefeat_body(ea_ref, eeW_ref, dW1e_ref, dW1a_ref, dW2_ref, euWe_ref,
                   wgs_ref, bias_ref, e_ref, env_ref):
    ea = ea_ref[...]
    el2 = jnp.sum(ea * ea, axis=1, keepdims=True)
    el = jnp.sqrt(el2)
    d = -0.75 / (el + 1e-8)
    step = (VMAX - VMIN) / (FC - 1)
    centers = VMIN + step * lax.broadcasted_iota(jnp.int32, (1, FC), 1).astype(jnp.float32)
    gamma = 1.0 / (step * step)
    diff = d - centers
    rbf = jnp.exp(-gamma * diff * diff)
    ef = _silu(_bdot(rbf, eeW_ref) + bias_ref[0:1, :])
    pre = (_bdot(ef, dW1e_ref) + _bdot(ea, dW1a_ref) + bias_ref[1:2, :])
    ef2 = _bdot(_silu(pre), dW2_ref) + bias_ref[2:3, :]
    e0 = _bdot(ef2, euWe_ref) + bias_ref[3:4, :]
    gate = _sigmoid(_bdot(e0, wgs_ref) + bias_ref[4:5, :])
    e1 = e0 + gate * e0
    mu = jnp.mean(e1, axis=1, keepdims=True)
    var = jnp.mean(e1 * e1, axis=1, keepdims=True) - mu * mu
    e2 = (e1 - mu) * lax.rsqrt(var + 1e-5) * bias_ref[5:6, :] + bias_ref[6:7, :]
    e_ref[...] = jnp.maximum(e2, 0.0)
    env = jnp.where(el < 5.0, jnp.cos(el * (jnp.pi / 10.0)) ** 2, 0.0)
    env_ref[...] = jnp.broadcast_to(env, env_ref.shape)


def _pack_proj(P):
    return jnp.concatenate(
        [_pack2(P[:, :128], P[:, 128:256]),
         _pack2(P[:, 256:384], P[:, 384:512])], axis=1)


def _proj0_body(x_ref, aW_ref, Wd_ref, Ws_ref, h_ref, Pd_ref, Ps_ref):
    h = _bdot(x_ref[...], aW_ref)
    h_ref[...] = h
    Pd_ref[...] = _pack_proj(_bdot(h, Wd_ref))
    Ps_ref[...] = _pack_proj(_bdot(h, Ws_ref))


def _projupd_body(h_ref, agg_ref, gb_ref, Wd_ref, Ws_ref,
                  hn_ref, Pd_ref, Ps_ref):
    agg = jnp.transpose(agg_ref[...])
    h = jnp.maximum(h_ref[...] + agg * gb_ref[0:1, :] + gb_ref[1:2, :], 0.0)
    hn_ref[...] = h
    Pd_ref[...] = _pack_proj(_bdot(h, Wd_ref))
    Ps_ref[...] = _pack_proj(_bdot(h, Ws_ref))


def _msg_body(Gd_ref, Gs_ref, e_ref, env_ref, Wf1e_ref, Wn1e_ref,
              Wf2_ref, Wn2_ref, bias_ref, msg_ref):
    e = e_ref[...]
    ef = _bdot(e, Wf1e_ref) + bias_ref[0:1, :]
    en = _bdot(e, Wn1e_ref) + bias_ref[2:3, :]
    sf = bias_ref[1:2, :]
    m = bias_ref[3:4, :]
    gdf = _unpack2(Gd_ref[:, :128])
    gsf = _unpack2(Gs_ref[:, :128])
    gdn = _unpack2(Gd_ref[:, 128:])
    gsn = _unpack2(Gs_ref[:, 128:])
    for j in range(2):
        pf = gdf[j] + gsf[j] + ef[:, j * 128:(j + 1) * 128]
        sf = sf + jnp.dot(_silu(pf).astype(jnp.bfloat16),
                          Wf2_ref[pl.ds(j * 128, 128), :],
                          preferred_element_type=jnp.float32)
        pn = gdn[j] + gsn[j] + en[:, j * 128:(j + 1) * 128]
        m = m + jnp.dot(_silu(pn).astype(jnp.bfloat16),
                        Wn2_ref[pl.ds(j * 128, 128), :],
                        preferred_element_type=jnp.float32)
    score = _sigmoid(sf * bias_ref[4:5, :] + bias_ref[5:6, :])
    env = env_ref[...][:, :1]
    msg_ref[...] = jnp.transpose(env * (score * m))


def _readout_body(h_ref, agg_ref, batch_ref, gb_ref, fcW_ref,
                  sums_ref, cnt_ref):
    i = pl.program_id(0)

    @pl.when(i == 0)
    def _():
        sums_ref[...] = jnp.zeros_like(sums_ref)
        cnt_ref[...] = jnp.zeros_like(cnt_ref)

    agg = jnp.transpose(agg_ref[...])
    h = jnp.maximum(h_ref[...] + agg * gb_ref[0:1, :] + gb_ref[1:2, :], 0.0)
    z = jnp.dot(h, fcW_ref[...], preferred_element_type=jnp.float32) + gb_ref[2:3, :]
    feat = jnp.maximum(z, 0.0) + jnp.log(1.0 + jnp.exp(-jnp.abs(z))) - 0.6931471805599453
    b0 = batch_ref[...][:, :1]
    gid = lax.broadcasted_iota(jnp.int32, (1, G), 1).astype(jnp.float32)
    mask = jnp.where(b0 == gid, 1.0, 0.0)
    part = lax.dot_general(mask, feat, (((0,), (0,)), ((), ())),
                           preferred_element_type=jnp.float32)
    ones = jnp.ones((mask.shape[0], 8), jnp.float32)
    cpart = lax.dot_general(mask, ones, (((0,), (0,)), ((), ())),
                            preferred_element_type=jnp.float32)
    sums_ref[...] += part
    cnt_ref[...] += cpart


def _final_body(sums_ref, cnt_ref, oW_ref, ob_ref, out_ref):
    cnt = jnp.maximum(cnt_ref[...][:, :1], 1.0)
    pooled = sums_ref[...] / cnt
    out_ref[...] = (jnp.dot(pooled, oW_ref[...], preferred_element_type=jnp.float32)
                    + ob_ref[...])


def _sc_gather_body(Pd_hbm, Ps_hbm, dst_hbm, src_hbm, Gd_hbm, Gs_hbm,
                    id0, id1, is0, is1, rowsd, rowss,
                    six0, six1, sgd, sgs, swd, sws):
    wid = lax.axis_index("s") * NC + lax.axis_index("c")
    nk = (NCH - 1 - wid) // NW + 1
    idd = (id0, id1)
    ids = (is0, is1)
    six = (six0, six1)

    def _issue_idx(k, p):
        base = (wid + k * NW) * CH
        pltpu.make_async_copy(dst_hbm.at[pl.ds(base, CH)], idd[p], six[p]).start()
        pltpu.make_async_copy(src_hbm.at[pl.ds(base, CH)], ids[p], six[p]).start()

    _issue_idx(0, 0)

    @pl.when(nk > 1)
    def _():
        _issue_idx(1, 1)

    def body(k, _):
        for p in range(2):
            kk = k * 2 + p

            @pl.when(kk < nk)
            def _():
                base = (wid + kk * NW) * CH
                pltpu.make_async_copy(dst_hbm.at[pl.ds(base, CH)], idd[p], six[p]).wait()
                pltpu.make_async_copy(src_hbm.at[pl.ds(base, CH)], ids[p], six[p]).wait()

                @pl.when(kk > 0)
                def _():
                    pltpu.make_async_copy(rowsd, Gd_hbm.at[pl.ds(base, CH)], swd).wait()
                    pltpu.make_async_copy(rowss, Gs_hbm.at[pl.ds(base, CH)], sws).wait()

                gd = pltpu.async_copy(Pd_hbm.at[idd[p]], rowsd, sgd)
                gs = pltpu.async_copy(Ps_hbm.at[ids[p]], rowss, sgs)

                @pl.when(kk + 2 < nk)
                def _():
                    _issue_idx(kk + 2, p)

                gd.wait()
                pltpu.make_async_copy(rowsd, Gd_hbm.at[pl.ds(base, CH)], swd).start()
                gs.wait()
                pltpu.make_async_copy(rowss, Gs_hbm.at[pl.ds(base, CH)], sws).start()
        return 0

    lax.fori_loop(0, (nk + 1) // 2, body, 0)
    last = (wid + (nk - 1) * NW) * CH
    pltpu.make_async_copy(rowsd, Gd_hbm.at[pl.ds(last, CH)], swd).wait()
    pltpu.make_async_copy(rowss, Gs_hbm.at[pl.ds(last, CH)], sws).wait()


def _sc_scatter_body(msg_hbm, dst_hbm, z_hbm, agg_hbm,
                     ib0, ib1, mb0, mb1, accum, si0, si1, sm0, sm1):
    wid = lax.axis_index("s") * NC + lax.axis_index("c")
    ct = wid * 8
    pltpu.sync_copy(z_hbm, accum)
    ibufs = (ib0, ib1)
    mbufs = (mb0, mb1)
    isems = (si0, si1)
    msems = (sm0, sm1)

    def _issue(k, b):
        base = k * CHS
        pltpu.make_async_copy(dst_hbm.at[pl.ds(base, CHS)],
                              ibufs[b], isems[b]).start()
        pltpu.make_async_copy(msg_hbm.at[pl.ds(ct, 8), pl.ds(base, CHS)],
                              mbufs[b], msems[b]).start()

    _issue(0, 0)
    _issue(1, 1)

    def outer(g, _):
        for b in range(2):
            k = g * 2 + b
            pltpu.make_async_copy(dst_hbm.at[pl.ds(k * CHS, CHS)],
                                  ibufs[b], isems[b]).wait()
            pltpu.make_async_copy(msg_hbm.at[pl.ds(ct, 8), pl.ds(k * CHS, CHS)],
                                  mbufs[b], msems[b]).wait()
            for j in range(CHS // 16):
                vidx = ibufs[b][pl.ds(j * 16, 16)]
                for c in range(8):
                    cfull = jnp.full((16,), c, jnp.int32)
                    val = mbufs[b][c, pl.ds(j * 16, 16)]
                    plsc.addupdate_scatter(accum, [cfull, vidx], val)

            @pl.when(k + 2 < NCHS)
            def _():
                _issue(k + 2, b)
        return 0

    lax.fori_loop(0, NCHS // 2, outer, 0)
    pltpu.sync_copy(accum, agg_hbm.at[pl.ds(ct, 8)])


def _sc_gather(Pd, Ps, dst, src):
    mesh = plsc.VectorSubcoreMesh(core_axis_name="c", subcore_axis_name="s",
                                  num_cores=NC, num_subcores=NS)
    f = pl.kernel(
        _sc_gather_body,
        out_type=[jax.ShapeDtypeStruct((E, FC), jnp.int32),
                  jax.ShapeDtypeStruct((E, FC), jnp.int32)],
        mesh=mesh,
        scratch_types=[pltpu.VMEM((CH,), jnp.int32),
                       pltpu.VMEM((CH,), jnp.int32),
                       pltpu.VMEM((CH,), jnp.int32),
                       pltpu.VMEM((CH,), jnp.int32),
                       pltpu.VMEM((CH, FC), jnp.int32),
                       pltpu.VMEM((CH, FC), jnp.int32),
                       pltpu.SemaphoreType.DMA,
                       pltpu.SemaphoreType.DMA,
                       pltpu.SemaphoreType.DMA,
                       pltpu.SemaphoreType.DMA,
                       pltpu.SemaphoreType.DMA,
                       pltpu.SemaphoreType.DMA],
    )
    return f(Pd, Ps, dst, src)


def _sc_scatter(msg, dst, z):
    mesh = plsc.VectorSubcoreMesh(core_axis_name="c", subcore_axis_name="s",
                                  num_cores=NC, num_subcores=NS)
    f = pl.kernel(
        _sc_scatter_body,
        out_type=jax.ShapeDtypeStruct((FC, NP), jnp.float32),
        mesh=mesh,
        scratch_types=[pltpu.VMEM((CHS,), jnp.int32),
                       pltpu.VMEM((CHS,), jnp.int32),
                       pltpu.VMEM((8, CHS), jnp.float32),
                       pltpu.VMEM((8, CHS), jnp.float32),
                       pltpu.VMEM((8, NP), jnp.float32),
                       pltpu.SemaphoreType.DMA,
                       pltpu.SemaphoreType.DMA,
                       pltpu.SemaphoreType.DMA,
                       pltpu.SemaphoreType.DMA],
        compiler_params=pltpu.CompilerParams(needs_layout_passes=False),
    )
    return f(msg, dst, z)


def _full(shape):
    return pl.BlockSpec(shape, lambda i: (0,) * len(shape))


def _edgefeat(ea8, eeW, dW1e, dW1a, dW2, euWe, wgs, bias):
    return _tc_call(
        _edgefeat_body,
        grid=(E // EB,),
        in_specs=[pl.BlockSpec((EB, 8), lambda i: (i, 0)),
                  _full((FC, FC)), _full((FC, FC)), _full((8, FC)),
                  _full((FC, FC)), _full((FC, FC)), _full((FC, FC)),
                  _full((8, FC))],
        out_specs=[pl.BlockSpec((EB, FC), lambda i: (i, 0)),
                   pl.BlockSpec((EB, 128), lambda i: (i, 0))],
        out_shape=[jax.ShapeDtypeStruct((E, FC), jnp.float32),
                   jax.ShapeDtypeStruct((E, 128), jnp.float32)],
    )(ea8, eeW, dW1e, dW1a, dW2, euWe, wgs, bias)


def _proj0(x_pad, aW, Wd, Ws):
    return _tc_call(
        _proj0_body,
        grid=(NP // RB,),
        in_specs=[pl.BlockSpec((RB, 128), lambda i: (i, 0)),
                  _full((128, FC)), _full((FC, 2 * FC)), _full((FC, 2 * FC))],
        out_specs=[pl.BlockSpec((RB, FC), lambda i: (i, 0)),
                   pl.BlockSpec((RB, FC), lambda i: (i, 0)),
                   pl.BlockSpec((RB, FC), lambda i: (i, 0))],
        out_shape=[jax.ShapeDtypeStruct((NP, FC), jnp.float32),
                   jax.ShapeDtypeStruct((NP, FC), jnp.int32),
                   jax.ShapeDtypeStruct((NP, FC), jnp.int32)],
    )(x_pad, aW, Wd, Ws)


def _projupd(h, agg, gb, Wd, Ws):
    return _tc_call(
        _projupd_body,
        grid=(NP // RB,),
        in_specs=[pl.BlockSpec((RB, FC), lambda i: (i, 0)),
                  pl.BlockSpec((FC, RB), lambda i: (0, i)),
                  _full((8, FC)), _full((FC, 2 * FC)), _full((FC, 2 * FC))],
        out_specs=[pl.BlockSpec((RB, FC), lambda i: (i, 0)),
                   pl.BlockSpec((RB, FC), lambda i: (i, 0)),
                   pl.BlockSpec((RB, FC), lambda i: (i, 0))],
        out_shape=[jax.ShapeDtypeStruct((NP, FC), jnp.float32),
                   jax.ShapeDtypeStruct((NP, FC), jnp.int32),
                   jax.ShapeDtypeStruct((NP, FC), jnp.int32)],
    )(h, agg, gb, Wd, Ws)


def _msg(Gd, Gs, e, env, Wf1e, Wn1e, Wf2, Wn2, bias):
    return _tc_call(
        _msg_body,
        grid=(E // EB,),
        in_specs=[pl.BlockSpec((EB, FC), lambda i: (i, 0)),
                  pl.BlockSpec((EB, FC), lambda i: (i, 0)),
                  pl.BlockSpec((EB, FC), lambda i: (i, 0)),
                  pl.BlockSpec((EB, 128), lambda i: (i, 0)),
                  _full((FC, FC)), _full((FC, FC)), _full((FC, FC)),
                  _full((FC, FC)), _full((8, FC))],
        out_specs=pl.BlockSpec((FC, EB), lambda i: (0, i)),
        out_shape=jax.ShapeDtypeStruct((FC, E), jnp.float32),
    )(Gd, Gs, e, env, Wf1e, Wn1e, Wf2, Wn2, bias)


def _readout(h, agg, batchf, gb, fcW):
    return _tc_call(
        _readout_body,
        grid=(NP // RB,),
        in_specs=[pl.BlockSpec((RB, FC), lambda i: (i, 0)),
                  pl.BlockSpec((FC, RB), lambda i: (0, i)),
                  pl.BlockSpec((RB, 8), lambda i: (i, 0)),
                  _full((8, FC)), _full((FC, FC))],
        out_specs=[pl.BlockSpec((G, FC), lambda i: (0, 0)),
                   pl.BlockSpec((G, 8), lambda i: (0, 0))],
        out_shape=[jax.ShapeDtypeStruct((G, FC), jnp.float32),
                   jax.ShapeDtypeStruct((G, 8), jnp.float32)],
    )(h, agg, batchf, gb, fcW)


def _final(sums, cnt, oW, ob):
    return _tc_call(
        _final_body,
        grid=(1,),
        in_specs=[_full((G, FC)), _full((G, 8)), _full((FC, 8)), _full((1, 8))],
        out_specs=_full((G, 8)),
        out_shape=jax.ShapeDtypeStruct((G, 8), jnp.float32),
    )(sums, cnt, oW, ob)


def kernel(x, edge_index, edge_attr, batch, params):
    p = params
    f32 = jnp.float32

    x_pad = jnp.zeros((NP, 128), f32).at[:N, :92].set(x)
    x_pad = x_pad.at[:N, 92].set(1.0)
    aW = jnp.zeros((128, FC), f32).at[:92, :].set(p['atom_W'])
    aW = aW.at[92, :].set(p['atom_b']).astype(jnp.bfloat16)

    ea8 = jnp.zeros((E, 8), f32).at[:, :3].set(edge_attr)
    dW1e = p['dir_W1'][:FC].astype(jnp.bfloat16)
    dW1a = jnp.zeros((8, FC), f32).at[:3, :].set(
        p['dir_W1'][FC:FC + 3]).astype(jnp.bfloat16)
    wgs = (p['eu_Wg'][:FC] + p['eu_Wg'][FC:]).astype(jnp.bfloat16)
    eeW = p['ee_W'].astype(jnp.bfloat16)
    dW2 = p['dir_W2'].astype(jnp.bfloat16)
    euWe = p['eu_We'].astype(jnp.bfloat16)
    bias_feat = jnp.stack([p['ee_b'], p['dir_b1'], p['dir_b2'], p['eu_be'],
                           p['eu_bg'], p['eu_ln_g'], p['eu_ln_b'],
                           jnp.zeros((FC,), f32)])

    src = edge_index[0].astype(jnp.int32)
    dst = edge_index[1].astype(jnp.int32)
    batchf = jnp.full((NP, 8), float(G), f32).at[:N, :].set(
        batch.astype(f32)[:, None])
    z_acc = jnp.zeros((8, NP), f32)

    Wd, Ws, Wf1e, Wn1e, Wf2, Wn2, bias_msg, gb_upd = [], [], [], [], [], [], [], []
    for l in range(LAYERS):
        wf1, wn1 = p['mp_Wf1'][l], p['mp_Wn1'][l]
        Wd.append(jnp.concatenate([wf1[:FC], wn1[:FC]],
                                  axis=1).astype(jnp.bfloat16))
        Ws.append(jnp.concatenate([wf1[FC:2 * FC], wn1[FC:2 * FC]],
                                  axis=1).astype(jnp.bfloat16))
        Wf1e.append(wf1[2 * FC:].astype(jnp.bfloat16))
        Wn1e.append(wn1[2 * FC:].astype(jnp.bfloat16))
        Wf2.append(p['mp_Wf2'][l].astype(jnp.bfloat16))
        Wn2.append(p['mp_Wn2'][l].astype(jnp.bfloat16))
        bias_msg.append(jnp.stack([p['mp_bf1'][l], p['mp_bf2'][l],
                                   p['mp_bn1'][l], p['mp_bn2'][l],
                                   p['mp_bni_g'][l], p['mp_bni_b'][l],
                                   jnp.zeros((FC,), f32), jnp.zeros((FC,), f32)]))
        gb_upd.append(jnp.stack([p['mp_bn_g'][l], p['mp_bn_b'][l], p['fc_b']]
                                + [jnp.zeros((FC,), f32)] * 5))

    oW = jnp.zeros((FC, 8), f32).at[:, 0].set(p['out_W'][:, 0])
    ob = jnp.zeros((1, 8), f32).at[0, 0].set(p['out_b'][0])

    e, env = _edgefeat(ea8, eeW, dW1e, dW1a, dW2, euWe, wgs, bias_feat)

    h, Pd, Ps = _proj0(x_pad, aW, Wd[0], Ws[0])
    for l in range(LAYERS):
        Gd, Gs = _sc_gather(Pd, Ps, dst, src)
        msg = _msg(Gd, Gs, e, env, Wf1e[l], Wn1e[l], Wf2[l], Wn2[l], bias_msg[l])
        agg = _sc_scatter(msg, dst, z_acc)
        if l < LAYERS - 1:
            h, Pd, Ps = _projupd(h, agg, gb_upd[l], Wd[l + 1], Ws[l + 1])

    sums, cnt = _readout(h, agg, batchf, gb_upd[LAYERS - 1], p['fc_W'])
    out8 = _final(sums, cnt, oW, ob)
    return out8[:, :1]


# speedup vs baseline: 1.0915x; 1.0915x over previous
import functools

import jax
import jax.numpy as jnp
from jax import lax
from jax.experimental import pallas as pl
from jax.experimental.pallas import tpu as pltpu
from jax.experimental.pallas import tpu_sc as plsc

FC = 256
N = 10000
E = 160000
G = 64
LAYERS = 3
VMIN, VMAX = -4.0, 4.0

NP = 10240
RB = 1024
EB = 1280
CH = 128
NCH = E // CH
CHS = 640
NCHS = E // CHS
NC, NS = 2, 16
NW = NC * NS

_tc_call = pl.pallas_call


def _silu(v):
    return v * (1.0 / (1.0 + jnp.exp(-v)))


def _pack2(a, b):
    ai = lax.bitcast_convert_type(a, jnp.int32) + 0x8000
    bi = lax.bitcast_convert_type(b, jnp.int32) + 0x8000
    return (ai & jnp.int32(-65536)) | (lax.shift_right_logical(bi, 16) & 0xFFFF)


def _unpack2(x):
    a = lax.bitcast_convert_type(x & jnp.int32(-65536), jnp.float32)
    b = lax.bitcast_convert_type(lax.shift_left(x, 16), jnp.float32)
    return a, b


def _sigmoid(v):
    return 1.0 / (1.0 + jnp.exp(-v))


def _bdot(a, w_ref):
    return jnp.dot(a, w_ref[...], preferred_element_type=jnp.float32)


def _edgefeat_body(ea_ref, eeW_ref, dW1e_ref, dW1a_ref, dW2_ref, euWe_ref,
                   wgs_ref, bias_ref, e_ref, env_ref):
    ea = ea_ref[...]
    el2 = jnp.sum(ea * ea, axis=1, keepdims=True)
    el = jnp.sqrt(el2)
    d = -0.75 / (el + 1e-8)
    step = (VMAX - VMIN) / (FC - 1)
    centers = VMIN + step * lax.broadcasted_iota(jnp.int32, (1, FC), 1).astype(jnp.float32)
    gamma = 1.0 / (step * step)
    diff = d - centers
    rbf = jnp.exp(-gamma * diff * diff)
    ef = _silu(_bdot(rbf, eeW_ref) + bias_ref[0:1, :])
    pre = (_bdot(ef, dW1e_ref) + _bdot(ea, dW1a_ref) + bias_ref[1:2, :])
    ef2 = _bdot(_silu(pre), dW2_ref) + bias_ref[2:3, :]
    e0 = _bdot(ef2, euWe_ref) + bias_ref[3:4, :]
    gate = _sigmoid(_bdot(e0, wgs_ref) + bias_ref[4:5, :])
    e1 = e0 + gate * e0
    mu = jnp.mean(e1, axis=1, keepdims=True)
    var = jnp.mean(e1 * e1, axis=1, keepdims=True) - mu * mu
    e2 = (e1 - mu) * lax.rsqrt(var + 1e-5) * bias_ref[5:6, :] + bias_ref[6:7, :]
    e_ref[...] = jnp.maximum(e2, 0.0)
    env = jnp.where(el < 5.0, jnp.cos(el * (jnp.pi / 10.0)) ** 2, 0.0)
    env_ref[...] = jnp.broadcast_to(env, env_ref.shape)


def _pack_proj(P):
    return jnp.concatenate(
        [_pack2(P[:, :128], P[:, 128:256]),
         _pack2(P[:, 256:384], P[:, 384:512])], axis=1)


def _proj0_body(x_ref, aW_ref, Wd_ref, Ws_ref, h_ref, Pd_ref, Ps_ref):
    h = _bdot(x_ref[...], aW_ref)
    h_ref[...] = h
    Pd_ref[...] = _pack_proj(_bdot(h, Wd_ref))
    Ps_ref[...] = _pack_proj(_bdot(h, Ws_ref))


def _projupd_body(h_ref, agg_ref, gb_ref, Wd_ref, Ws_ref,
                  hn_ref, Pd_ref, Ps_ref):
    agg = jnp.transpose(agg_ref[...])
    h = jnp.maximum(h_ref[...] + agg * gb_ref[0:1, :] + gb_ref[1:2, :], 0.0)
    hn_ref[...] = h
    Pd_ref[...] = _pack_proj(_bdot(h, Wd_ref))
    Ps_ref[...] = _pack_proj(_bdot(h, Ws_ref))


def _msg_body(Gd_ref, Gs_ref, e_ref, env_ref, Wf1e_ref, Wn1e_ref,
              Wf2_ref, Wn2_ref, bias_ref, msg_ref):
    e = e_ref[...]
    ef = _bdot(e, Wf1e_ref) + bias_ref[0:1, :]
    en = _bdot(e, Wn1e_ref) + bias_ref[2:3, :]
    sf = bias_ref[1:2, :]
    m = bias_ref[3:4, :]
    gdf = _unpack2(Gd_ref[:, :128])
    gsf = _unpack2(Gs_ref[:, :128])
    gdn = _unpack2(Gd_ref[:, 128:])
    gsn = _unpack2(Gs_ref[:, 128:])
    for j in range(2):
        pf = gdf[j] + gsf[j] + ef[:, j * 128:(j + 1) * 128]
        sf = sf + jnp.dot(_silu(pf), Wf2_ref[pl.ds(j * 128, 128), :],
                          preferred_element_type=jnp.float32)
        pn = gdn[j] + gsn[j] + en[:, j * 128:(j + 1) * 128]
        m = m + jnp.dot(_silu(pn), Wn2_ref[pl.ds(j * 128, 128), :],
                        preferred_element_type=jnp.float32)
    score = _sigmoid(sf * bias_ref[4:5, :] + bias_ref[5:6, :])
    env = env_ref[...][:, :1]
    msg_ref[...] = jnp.transpose(env * (score * m))


def _readout_body(h_ref, agg_ref, batch_ref, gb_ref, fcW_ref,
                  sums_ref, cnt_ref):
    i = pl.program_id(0)

    @pl.when(i == 0)
    def _():
        sums_ref[...] = jnp.zeros_like(sums_ref)
        cnt_ref[...] = jnp.zeros_like(cnt_ref)

    agg = jnp.transpose(agg_ref[...])
    h = jnp.maximum(h_ref[...] + agg * gb_ref[0:1, :] + gb_ref[1:2, :], 0.0)
    z = jnp.dot(h, fcW_ref[...], preferred_element_type=jnp.float32) + gb_ref[2:3, :]
    feat = jnp.maximum(z, 0.0) + jnp.log(1.0 + jnp.exp(-jnp.abs(z))) - 0.6931471805599453
    b0 = batch_ref[...][:, :1]
    gid = lax.broadcasted_iota(jnp.int32, (1, G), 1).astype(jnp.float32)
    mask = jnp.where(b0 == gid, 1.0, 0.0)
    part = lax.dot_general(mask, feat, (((0,), (0,)), ((), ())),
                           preferred_element_type=jnp.float32)
    ones = jnp.ones((mask.shape[0], 8), jnp.float32)
    cpart = lax.dot_general(mask, ones, (((0,), (0,)), ((), ())),
                            preferred_element_type=jnp.float32)
    sums_ref[...] += part
    cnt_ref[...] += cpart


def _final_body(sums_ref, cnt_ref, oW_ref, ob_ref, out_ref):
    cnt = jnp.maximum(cnt_ref[...][:, :1], 1.0)
    pooled = sums_ref[...] / cnt
    out_ref[...] = (jnp.dot(pooled, oW_ref[...], preferred_element_type=jnp.float32)
                    + ob_ref[...])


def _sc_gather_body(Pd_hbm, Ps_hbm, dst_hbm, src_hbm, Gd_hbm, Gs_hbm,
                    id0, id1, is0, is1, rowsd, rowss,
                    six0, six1, sgd, sgs, swd, sws):
    wid = lax.axis_index("s") * NC + lax.axis_index("c")
    nk = (NCH - 1 - wid) // NW + 1
    idd = (id0, id1)
    ids = (is0, is1)
    six = (six0, six1)

    def _issue_idx(k, p):
        base = (wid + k * NW) * CH
        pltpu.make_async_copy(dst_hbm.at[pl.ds(base, CH)], idd[p], six[p]).start()
        pltpu.make_async_copy(src_hbm.at[pl.ds(base, CH)], ids[p], six[p]).start()

    _issue_idx(0, 0)

    @pl.when(nk > 1)
    def _():
        _issue_idx(1, 1)

    def body(k, _):
        for p in range(2):
            kk = k * 2 + p

            @pl.when(kk < nk)
            def _():
                base = (wid + kk * NW) * CH
                pltpu.make_async_copy(dst_hbm.at[pl.ds(base, CH)], idd[p], six[p]).wait()
                pltpu.make_async_copy(src_hbm.at[pl.ds(base, CH)], ids[p], six[p]).wait()

                @pl.when(kk > 0)
                def _():
                    pltpu.make_async_copy(rowsd, Gd_hbm.at[pl.ds(base, CH)], swd).wait()
                    pltpu.make_async_copy(rowss, Gs_hbm.at[pl.ds(base, CH)], sws).wait()

                gd = pltpu.async_copy(Pd_hbm.at[idd[p]], rowsd, sgd)
                gs = pltpu.async_copy(Ps_hbm.at[ids[p]], rowss, sgs)

                @pl.when(kk + 2 < nk)
                def _():
                    _issue_idx(kk + 2, p)

                gd.wait()
                pltpu.make_async_copy(rowsd, Gd_hbm.at[pl.ds(base, CH)], swd).start()
                gs.wait()
                pltpu.make_async_copy(rowss, Gs_hbm.at[pl.ds(base, CH)], sws).start()
        return 0

    lax.fori_loop(0, (nk + 1) // 2, body, 0)
    last = (wid + (nk - 1) * NW) * CH
    pltpu.make_async_copy(rowsd, Gd_hbm.at[pl.ds(last, CH)], swd).wait()
    pltpu.make_async_copy(rowss, Gs_hbm.at[pl.ds(last, CH)], sws).wait()


def _sc_scatter_body(msg_hbm, dst_hbm, z_hbm, agg_hbm,
                     ib0, ib1, mb0, mb1, accum, si0, si1, sm0, sm1):
    wid = lax.axis_index("s") * NC + lax.axis_index("c")
    ct = wid * 8
    pltpu.sync_copy(z_hbm, accum)
    ibufs = (ib0, ib1)
    mbufs = (mb0, mb1)
    isems = (si0, si1)
    msems = (sm0, sm1)

    def _issue(k, b):
        base = k * CHS
        pltpu.make_async_copy(dst_hbm.at[pl.ds(base, CHS)],
                              ibufs[b], isems[b]).start()
        pltpu.make_async_copy(msg_hbm.at[pl.ds(ct, 8), pl.ds(base, CHS)],
                              mbufs[b], msems[b]).start()

    _issue(0, 0)
    _issue(1, 1)

    def outer(g, _):
        for b in range(2):
            k = g * 2 + b
            pltpu.make_async_copy(dst_hbm.at[pl.ds(k * CHS, CHS)],
                                  ibufs[b], isems[b]).wait()
            pltpu.make_async_copy(msg_hbm.at[pl.ds(ct, 8), pl.ds(k * CHS, CHS)],
                                  mbufs[b], msems[b]).wait()
            for j in range(CHS // 16):
                vidx = ibufs[b][pl.ds(j * 16, 16)]
                for c in range(8):
                    cfull = jnp.full((16,), c, jnp.int32)
                    val = mbufs[b][c, pl.ds(j * 16, 16)]
                    plsc.addupdate_scatter(accum, [cfull, vidx], val)

            @pl.when(k + 2 < NCHS)
            def _():
                _issue(k + 2, b)
        return 0

    lax.fori_loop(0, NCHS // 2, outer, 0)
    pltpu.sync_copy(accum, agg_hbm.at[pl.ds(ct, 8)])


def _sc_gather(Pd, Ps, dst, src):
    mesh = plsc.VectorSubcoreMesh(core_axis_name="c", subcore_axis_name="s",
                                  num_cores=NC, num_subcores=NS)
    f = pl.kernel(
        _sc_gather_body,
        out_type=[jax.ShapeDtypeStruct((E, FC), jnp.int32),
                  jax.ShapeDtypeStruct((E, FC), jnp.int32)],
        mesh=mesh,
        scratch_types=[pltpu.VMEM((CH,), jnp.int32),
                       pltpu.VMEM((CH,), jnp.int32),
                       pltpu.VMEM((CH,), jnp.int32),
                       pltpu.VMEM((CH,), jnp.int32),
                       pltpu.VMEM((CH, FC), jnp.int32),
                       pltpu.VMEM((CH, FC), jnp.int32),
                       pltpu.SemaphoreType.DMA,
                       pltpu.SemaphoreType.DMA,
                       pltpu.SemaphoreType.DMA,
                       pltpu.SemaphoreType.DMA,
                       pltpu.SemaphoreType.DMA,
                       pltpu.SemaphoreType.DMA],
    )
    return f(Pd, Ps, dst, src)


def _sc_scatter(msg, dst, z):
    mesh = plsc.VectorSubcoreMesh(core_axis_name="c", subcore_axis_name="s",
                                  num_cores=NC, num_subcores=NS)
    f = pl.kernel(
        _sc_scatter_body,
        out_type=jax.ShapeDtypeStruct((FC, NP), jnp.float32),
        mesh=mesh,
        scratch_types=[pltpu.VMEM((CHS,), jnp.int32),
                       pltpu.VMEM((CHS,), jnp.int32),
                       pltpu.VMEM((8, CHS), jnp.float32),
                       pltpu.VMEM((8, CHS), jnp.float32),
                       pltpu.VMEM((8, NP), jnp.float32),
                       pltpu.SemaphoreType.DMA,
                       pltpu.SemaphoreType.DMA,
                       pltpu.SemaphoreType.DMA,
                       pltpu.SemaphoreType.DMA],
        compiler_params=pltpu.CompilerParams(needs_layout_passes=False),
    )
    return f(msg, dst, z)


def _full(shape):
    return pl.BlockSpec(shape, lambda i: (0,) * len(shape))


def _edgefeat(ea8, eeW, dW1e, dW1a, dW2, euWe, wgs, bias):
    return _tc_call(
        _edgefeat_body,
        grid=(E // EB,),
        in_specs=[pl.BlockSpec((EB, 8), lambda i: (i, 0)),
                  _full((FC, FC)), _full((FC, FC)), _full((8, FC)),
                  _full((FC, FC)), _full((FC, FC)), _full((FC, FC)),
                  _full((8, FC))],
        out_specs=[pl.BlockSpec((EB, FC), lambda i: (i, 0)),
                   pl.BlockSpec((EB, 128), lambda i: (i, 0))],
        out_shape=[jax.ShapeDtypeStruct((E, FC), jnp.float32),
                   jax.ShapeDtypeStruct((E, 128), jnp.float32)],
    )(ea8, eeW, dW1e, dW1a, dW2, euWe, wgs, bias)


def _proj0(x_pad, aW, Wd, Ws):
    return _tc_call(
        _proj0_body,
        grid=(NP // RB,),
        in_specs=[pl.BlockSpec((RB, 128), lambda i: (i, 0)),
                  _full((128, FC)), _full((FC, 2 * FC)), _full((FC, 2 * FC))],
        out_specs=[pl.BlockSpec((RB, FC), lambda i: (i, 0)),
                   pl.BlockSpec((RB, FC), lambda i: (i, 0)),
                   pl.BlockSpec((RB, FC), lambda i: (i, 0))],
        out_shape=[jax.ShapeDtypeStruct((NP, FC), jnp.float32),
                   jax.ShapeDtypeStruct((NP, FC), jnp.int32),
                   jax.ShapeDtypeStruct((NP, FC), jnp.int32)],
    )(x_pad, aW, Wd, Ws)


def _projupd(h, agg, gb, Wd, Ws):
    return _tc_call(
        _projupd_body,
        grid=(NP // RB,),
        in_specs=[pl.BlockSpec((RB, FC), lambda i: (i, 0)),
                  pl.BlockSpec((FC, RB), lambda i: (0, i)),
                  _full((8, FC)), _full((FC, 2 * FC)), _full((FC, 2 * FC))],
        out_specs=[pl.BlockSpec((RB, FC), lambda i: (i, 0)),
                   pl.BlockSpec((RB, FC), lambda i: (i, 0)),
                   pl.BlockSpec((RB, FC), lambda i: (i, 0))],
        out_shape=[jax.ShapeDtypeStruct((NP, FC), jnp.float32),
                   jax.ShapeDtypeStruct((NP, FC), jnp.int32),
                   jax.ShapeDtypeStruct((NP, FC), jnp.int32)],
    )(h, agg, gb, Wd, Ws)


def _msg(Gd, Gs, e, env, Wf1e, Wn1e, Wf2, Wn2, bias):
    return _tc_call(
        _msg_body,
        grid=(E // EB,),
        in_specs=[pl.BlockSpec((EB, FC), lambda i: (i, 0)),
                  pl.BlockSpec((EB, FC), lambda i: (i, 0)),
                  pl.BlockSpec((EB, FC), lambda i: (i, 0)),
                  pl.BlockSpec((EB, 128), lambda i: (i, 0)),
                  _full((FC, FC)), _full((FC, FC)), _full((FC, FC)),
                  _full((FC, FC)), _full((8, FC))],
        out_specs=pl.BlockSpec((FC, EB), lambda i: (0, i)),
        out_shape=jax.ShapeDtypeStruct((FC, E), jnp.float32),
    )(Gd, Gs, e, env, Wf1e, Wn1e, Wf2, Wn2, bias)


def _readout(h, agg, batchf, gb, fcW):
    return _tc_call(
        _readout_body,
        grid=(NP // RB,),
        in_specs=[pl.BlockSpec((RB, FC), lambda i: (i, 0)),
                  pl.BlockSpec((FC, RB), lambda i: (0, i)),
                  pl.BlockSpec((RB, 8), lambda i: (i, 0)),
                  _full((8, FC)), _full((FC, FC))],
        out_specs=[pl.BlockSpec((G, FC), lambda i: (0, 0)),
                   pl.BlockSpec((G, 8), lambda i: (0, 0))],
        out_shape=[jax.ShapeDtypeStruct((G, FC), jnp.float32),
                   jax.ShapeDtypeStruct((G, 8), jnp.float32)],
    )(h, agg, batchf, gb, fcW)


def _final(sums, cnt, oW, ob):
    return _tc_call(
        _final_body,
        grid=(1,),
        in_specs=[_full((G, FC)), _full((G, 8)), _full((FC, 8)), _full((1, 8))],
        out_specs=_full((G, 8)),
        out_shape=jax.ShapeDtypeStruct((G, 8), jnp.float32),
    )(sums, cnt, oW, ob)


def kernel(x, edge_index, edge_attr, batch, params):
    p = params
    f32 = jnp.float32

    x_pad = jnp.zeros((NP, 128), f32).at[:N, :92].set(x)
    x_pad = x_pad.at[:N, 92].set(1.0)
    aW = jnp.zeros((128, FC), f32).at[:92, :].set(p['atom_W'])
    aW = aW.at[92, :].set(p['atom_b'])

    ea8 = jnp.zeros((E, 8), f32).at[:, :3].set(edge_attr)
    dW1e = p['dir_W1'][:FC]
    dW1a = jnp.zeros((8, FC), f32).at[:3, :].set(p['dir_W1'][FC:FC + 3])
    wgs = p['eu_Wg'][:FC] + p['eu_Wg'][FC:]
    eeW = p['ee_W']
    dW2 = p['dir_W2']
    euWe = p['eu_We']
    bias_feat = jnp.stack([p['ee_b'], p['dir_b1'], p['dir_b2'], p['eu_be'],
                           p['eu_bg'], p['eu_ln_g'], p['eu_ln_b'],
                           jnp.zeros((FC,), f32)])

    src = edge_index[0].astype(jnp.int32)
    dst = edge_index[1].astype(jnp.int32)
    batchf = jnp.full((NP, 8), float(G), f32).at[:N, :].set(
        batch.astype(f32)[:, None])
    z_acc = jnp.zeros((8, NP), f32)

    Wd, Ws, Wf1e, Wn1e, Wf2, Wn2, bias_msg, gb_upd = [], [], [], [], [], [], [], []
    for l in range(LAYERS):
        wf1, wn1 = p['mp_Wf1'][l], p['mp_Wn1'][l]
        Wd.append(jnp.concatenate([wf1[:FC], wn1[:FC]], axis=1))
        Ws.append(jnp.concatenate([wf1[FC:2 * FC], wn1[FC:2 * FC]], axis=1))
        Wf1e.append(wf1[2 * FC:])
        Wn1e.append(wn1[2 * FC:])
        Wf2.append(p['mp_Wf2'][l])
        Wn2.append(p['mp_Wn2'][l])
        bias_msg.append(jnp.stack([p['mp_bf1'][l], p['mp_bf2'][l],
                                   p['mp_bn1'][l], p['mp_bn2'][l],
                                   p['mp_bni_g'][l], p['mp_bni_b'][l],
                                   jnp.zeros((FC,), f32), jnp.zeros((FC,), f32)]))
        gb_upd.append(jnp.stack([p['mp_bn_g'][l], p['mp_bn_b'][l], p['fc_b']]
                                + [jnp.zeros((FC,), f32)] * 5))

    oW = jnp.zeros((FC, 8), f32).at[:, 0].set(p['out_W'][:, 0])
    ob = jnp.zeros((1, 8), f32).at[0, 0].set(p['out_b'][0])

    e, env = _edgefeat(ea8, eeW, dW1e, dW1a, dW2, euWe, wgs, bias_feat)

    h, Pd, Ps = _proj0(x_pad, aW, Wd[0], Ws[0])
    for l in range(LAYERS):
        Gd, Gs = _sc_gather(Pd, Ps, dst, src)
        msg = _msg(Gd, Gs, e, env, Wf1e[l], Wn1e[l], Wf2[l], Wn2[l], bias_msg[l])
        agg = _sc_scatter(msg, dst, z_acc)
        if l < LAYERS - 1:
            h, Pd, Ps = _projupd(h, agg, gb_upd[l], Wd[l + 1], Ws[l + 1])

    sums, cnt = _readout(h, agg, batchf, gb_upd[LAYERS - 1], p['fc_W'])
    out8 = _final(sums, cnt, oW, ob)
    return out8[:, :1]
